# baseline (speedup 1.0000x reference)
import functools

import jax
import jax.numpy as jnp
from jax import lax
from jax.experimental import pallas as pl
from jax.experimental.pallas import tpu as pltpu
from jax.experimental.pallas import tpu_sc as plsc

_LANES = 16


@functools.lru_cache(maxsize=None)
def _make_pool_kernel(V, D, B, H, NB=4):
    info = plsc.get_sparse_core_info()
    NC, NS = info.num_cores, info.num_subcores
    NW = NC * NS
    assert B % NW == 0 and D % _LANES == 0 and H % 8 == 0
    b_per_w = B // NW
    assert b_per_w % NB == 0
    n_idx = b_per_w * H
    chunks = []
    off = 0
    while off < H:
        ln = min(128, H - off)
        chunks.append((off, ln))
        off += ln

    mesh = plsc.VectorSubcoreMesh(core_axis_name="c", subcore_axis_name="s")

    @functools.partial(
        pl.kernel,
        out_type=jax.ShapeDtypeStruct((B, D), jnp.float32),
        mesh=mesh,
        scratch_types=[
            pltpu.VMEM((n_idx,), jnp.int32),
            pltpu.VMEM((NB, H, D), jnp.float32),
            pltpu.VMEM((b_per_w, D), jnp.float32),
        ] + [pltpu.SemaphoreType.DMA] * NB,
        compiler_params=pltpu.CompilerParams(use_tc_tiling_on_sc=False),
        name="sc_embed_sum_pool",
    )
    def pool_kernel(x_hbm, table_hbm, out_hbm, idx_v, rows_v, pooled_v, *sems):
        wid = lax.axis_index("s") * NC + lax.axis_index("c")
        base = wid * b_per_w
        pltpu.sync_copy(x_hbm.at[pl.ds(base * H, n_idx)], idx_v)

        def gather_descs(e, k):
            return [
                pltpu.make_async_copy(
                    table_hbm.at[idx_v.at[pl.ds(e * H + off, ln)]],
                    rows_v.at[k].at[pl.ds(off, ln)],
                    sems[k],
                )
                for off, ln in chunks
            ]

        for k in range(NB):
            for d_ in gather_descs(k, k):
                d_.start()

        def do_elem(e, k):
            for d_ in gather_descs(e, k):
                d_.wait()

            def inner(j, accs):
                return tuple(
                    accs[d] + rows_v[k, j, pl.ds(d * _LANES, _LANES)]
                    for d in range(D // _LANES)
                )

            zeros = tuple(
                jnp.zeros((_LANES,), jnp.float32) for _ in range(D // _LANES)
            )
            accs = lax.fori_loop(0, H, inner, zeros, unroll=4)
            for d in range(D // _LANES):
                pooled_v[e, pl.ds(d * _LANES, _LANES)] = accs[d]

            @pl.when(e + NB < b_per_w)
            def _():
                for d_ in gather_descs(e + NB, k):
                    d_.start()

        def body(i, carry):
            for k in range(NB):
                do_elem(i * NB + k, k)
            return carry

        lax.fori_loop(0, b_per_w // NB, body, 0)
        pltpu.sync_copy(pooled_v, out_hbm.at[pl.ds(base, b_per_w)])

    return pool_kernel


def _tc_pack_body(tt_ref, o_ref):
    t = jnp.swapaxes(tt_ref[...], 0, 1)
    t3 = t.reshape(t.shape[0] // 2, 2, t.shape[1])
    o_ref[...] = jnp.concatenate([t3[:, 0, :], t3[:, 1, :]], axis=1)


@functools.lru_cache(maxsize=None)
def _make_tc_pack_kernel(V, D, blk):
    n = (V + blk - 1) // blk
    return pl.pallas_call(
        _tc_pack_body,
        grid=(n,),
        in_specs=[pl.BlockSpec((D, blk), lambda j: (0, j))],
        out_specs=pl.BlockSpec((blk // 2, 2 * D), lambda j: (j, 0)),
        out_shape=jax.ShapeDtypeStruct((V // 2, 2 * D), jnp.float32),
    )


def _head_body(p_ref, w_ref, b_ref, o_ref):
    o_ref[...] = (
        lax.dot_general(
            p_ref[...], w_ref[...],
            dimension_numbers=(((1,), (1,)), ((), ())),
            preferred_element_type=jnp.float32,
        )
        + b_ref[...]
    )


@functools.lru_cache(maxsize=None)
def _make_head_kernel(B, D, C):
    return pl.pallas_call(
        _head_body,
        out_shape=jax.ShapeDtypeStruct((B, C), jnp.float32),
    )


def kernel(x, table, W, b):
    B, H = x.shape
    V, D = table.shape
    C = W.shape[0]
    x_flat = x.reshape(B * H).astype(jnp.int32)
    packed = _make_tc_pack_kernel(V, D, 16384)(table.T)
    pooled = _make_pool_kernel(V, D, B, H)(x_flat, packed.reshape(V, D))
    return _make_head_kernel(B, D, C)(pooled, W, b.reshape(1, C))


# speedup vs baseline: 1.0008x; 1.0008x over previous
import functools

import jax
import jax.numpy as jnp
from jax import lax
from jax.experimental import pallas as pl
from jax.experimental.pallas import tpu as pltpu
from jax.experimental.pallas import tpu_sc as plsc

_LANES = 16


@functools.lru_cache(maxsize=None)
def _make_pool_kernel(V, D, B, H, NB=4):
    info = plsc.get_sparse_core_info()
    NC, NS = info.num_cores, info.num_subcores
    NW = NC * NS
    assert B % NW == 0 and D % _LANES == 0 and H % 8 == 0
    b_per_w = B // NW
    assert b_per_w % NB == 0
    n_idx = b_per_w * H
    chunks = []
    off = 0
    while off < H:
        ln = min(128, H - off)
        chunks.append((off, ln))
        off += ln

    mesh = plsc.VectorSubcoreMesh(core_axis_name="c", subcore_axis_name="s")

    @functools.partial(
        pl.kernel,
        out_type=jax.ShapeDtypeStruct((B, D), jnp.float32),
        mesh=mesh,
        scratch_types=[
            pltpu.VMEM((n_idx,), jnp.int32),
            pltpu.VMEM((NB, H, D), jnp.float32),
            pltpu.VMEM((b_per_w, D), jnp.float32),
        ] + [pltpu.SemaphoreType.DMA] * NB,
        compiler_params=pltpu.CompilerParams(use_tc_tiling_on_sc=False),
        name="sc_embed_sum_pool",
    )
    def pool_kernel(x_hbm, table_hbm, out_hbm, idx_v, rows_v, pooled_v, *sems):
        wid = lax.axis_index("s") * NC + lax.axis_index("c")
        base = wid * b_per_w
        pltpu.sync_copy(x_hbm.at[pl.ds(base * H, n_idx)], idx_v)

        def gather_descs(e, k):
            return [
                pltpu.make_async_copy(
                    table_hbm.at[idx_v.at[pl.ds(e * H + off, ln)]],
                    rows_v.at[k].at[pl.ds(off, ln)],
                    sems[k],
                )
                for off, ln in chunks
            ]

        for k in range(NB):
            for d_ in gather_descs(k, k):
                d_.start()

        def do_elem(e, k):
            for d_ in gather_descs(e, k):
                d_.wait()

            def inner(j, accs):
                return tuple(
                    accs[d] + rows_v[k, j, pl.ds(d * _LANES, _LANES)]
                    for d in range(D // _LANES)
                )

            zeros = tuple(
                jnp.zeros((_LANES,), jnp.float32) for _ in range(D // _LANES)
            )
            accs = lax.fori_loop(0, H, inner, zeros, unroll=8)
            for d in range(D // _LANES):
                pooled_v[e, pl.ds(d * _LANES, _LANES)] = accs[d]

            @pl.when(e + NB < b_per_w)
            def _():
                for d_ in gather_descs(e + NB, k):
                    d_.start()

        def body(i, carry):
            for k in range(NB):
                do_elem(i * NB + k, k)
            return carry

        lax.fori_loop(0, b_per_w // NB, body, 0)
        pltpu.sync_copy(pooled_v, out_hbm.at[pl.ds(base, b_per_w)])

    return pool_kernel


def _tc_pack_body(tt_ref, o_ref):
    t = jnp.swapaxes(tt_ref[...], 0, 1)
    t3 = t.reshape(t.shape[0] // 2, 2, t.shape[1])
    o_ref[...] = jnp.concatenate([t3[:, 0, :], t3[:, 1, :]], axis=1)


@functools.lru_cache(maxsize=None)
def _make_tc_pack_kernel(V, D, blk):
    n = (V + blk - 1) // blk
    return pl.pallas_call(
        _tc_pack_body,
        grid=(n,),
        in_specs=[pl.BlockSpec((D, blk), lambda j: (0, j))],
        out_specs=pl.BlockSpec((blk // 2, 2 * D), lambda j: (j, 0)),
        out_shape=jax.ShapeDtypeStruct((V // 2, 2 * D), jnp.float32),
    )


def _head_body(p_ref, w_ref, b_ref, o_ref):
    o_ref[...] = (
        lax.dot_general(
            p_ref[...], w_ref[...],
            dimension_numbers=(((1,), (1,)), ((), ())),
            preferred_element_type=jnp.float32,
        )
        + b_ref[...]
    )


@functools.lru_cache(maxsize=None)
def _make_head_kernel(B, D, C):
    return pl.pallas_call(
        _head_body,
        out_shape=jax.ShapeDtypeStruct((B, C), jnp.float32),
    )


def kernel(x, table, W, b):
    B, H = x.shape
    V, D = table.shape
    C = W.shape[0]
    x_flat = x.reshape(B * H).astype(jnp.int32)
    packed = _make_tc_pack_kernel(V, D, 16384)(table.T)
    pooled = _make_pool_kernel(V, D, B, H)(x_flat, packed.reshape(V, D))
    return _make_head_kernel(B, D, C)(pooled, W, b.reshape(1, C))
